# baseline (speedup 1.0000x reference)
import functools

import jax
import jax.numpy as jnp
from jax.experimental import pallas as pl
from jax.experimental.pallas import tpu as pltpu

_K = 32
_WIN = 128
_TB = 256
_EB = 512
_NEG = -1e9
_R = 8

_HI = jax.lax.Precision.HIGHEST


def _bdot(a, b):
    return jax.lax.dot_general(
        a.astype(jnp.bfloat16), b.astype(jnp.bfloat16),
        (((1,), (0,)), ((), ())), preferred_element_type=jnp.float32)


def _bdotT(a, b):
    return jax.lax.dot_general(
        a.astype(jnp.bfloat16), b.astype(jnp.bfloat16),
        (((1,), (1,)), ((), ())), preferred_element_type=jnp.float32)


def _fdot(a, b):
    return jax.lax.dot_general(a, b, (((1,), (0,)), ((), ())),
                               preferred_element_type=jnp.float32)


def _xdot(a, b):
    return jax.lax.dot_general(a, b, (((1,), (0,)), ((), ())),
                               preferred_element_type=jnp.float32,
                               precision=_HI)


def _xdotT(a, b):
    return jax.lax.dot_general(a, b, (((1,), (1,)), ((), ())),
                               preferred_element_type=jnp.float32,
                               precision=_HI)


def _xdot0(a, b):
    return jax.lax.dot_general(a, b, (((0,), (0,)), ((), ())),
                               preferred_element_type=jnp.float32,
                               precision=_HI)


def _rms(x, g):
    return x * g / jnp.sqrt(jnp.mean(x * x, axis=-1, keepdims=True) + 1e-6)


def _topk_into(s_col, e_ref, v_ref, k):
    S = s_col.shape[0]
    iota_col = jax.lax.broadcasted_iota(jnp.int32, (S, 1), 0).astype(jnp.float32)
    iota_row = jax.lax.broadcasted_iota(jnp.int32, (1, S), 1).astype(jnp.float32)

    def body(j, s):
        m = jnp.max(s)
        idx = jnp.min(jnp.where(s == m, iota_col, jnp.float32(S)))
        e_ref[pl.ds(j, 1), :] = (iota_row == idx).astype(jnp.float32)
        v_ref[pl.ds(j, 1), :] = jnp.reshape(m, (1, 1))
        return jnp.where(iota_col == idx, -jnp.inf, s)

    jax.lax.fori_loop(0, k, body, s_col)


def _embed_body(ids_ref, *refs):
    out_ref = refs[-1]
    for j in range(_R):
        out_ref[j, :] = refs[j][0, 0, :]


def _embed_gather(ids_flat, embed):
    T = ids_flat.shape[0]
    V, D = embed.shape
    embed3 = embed.reshape(V, 1, D)

    def imap(j, i, ids):
        return (ids[i * _R + j], 0, 0)

    return pl.pallas_call(
        _embed_body,
        grid_spec=pltpu.PrefetchScalarGridSpec(
            num_scalar_prefetch=1,
            grid=(T // _R,),
            in_specs=[pl.BlockSpec((1, 1, D), functools.partial(imap, j))
                      for j in range(_R)],
            out_specs=pl.BlockSpec((_R, D), lambda i, ids: (i, 0)),
        ),
        out_shape=jax.ShapeDtypeStruct((T, D), jnp.float32),
    )(ids_flat, *([embed3] * _R))


def _kvc_body(enc_ref, wkv_ref, o_ref):
    o_ref[...] = _bdot(enc_ref[...], wkv_ref[...]).astype(jnp.bfloat16)


def _kvc_proj(enc_bf, wkv_bf):
    T, D = enc_bf.shape
    D2 = wkv_bf.shape[1]
    eb = min(_EB, T)
    return pl.pallas_call(
        _kvc_body,
        grid=(T // eb,),
        in_specs=[
            pl.BlockSpec((eb, D), lambda i: (i, 0)),
            pl.BlockSpec(wkv_bf.shape, lambda i: (0, 0)),
        ],
        out_specs=pl.BlockSpec((eb, D2), lambda i: (i, 0)),
        out_shape=jax.ShapeDtypeStruct((T, D2), jnp.bfloat16),
    )(enc_bf, wkv_bf)


def _light_body(x_ref, ga_ref, wqkv_ref, wo_ref, o_ref):
    x = x_ref[...]
    dl = wo_ref.shape[0]
    xn = _rms(x, ga_ref[...])
    qkv = _bdot(xn, wqkv_ref[...])
    q = qkv[:, :dl]
    k = qkv[:, dl:2 * dl]
    v = qkv[:, 2 * dl:]
    a = _bdotT(q, k) / (dl ** 0.5)
    W = x.shape[0]
    r = jax.lax.broadcasted_iota(jnp.int32, (W, W), 0)
    c = jax.lax.broadcasted_iota(jnp.int32, (W, W), 1)
    a = jax.nn.softmax(jnp.where(r >= c, a, _NEG), axis=-1)
    o_ref[...] = x + _bdot(_bdot(a, v), wo_ref[...])


def _light_attn(x, ga, wqkv, wo):
    T, D = x.shape
    return pl.pallas_call(
        _light_body,
        grid=(T // _WIN,),
        in_specs=[
            pl.BlockSpec((_WIN, D), lambda i: (i, 0)),
            pl.BlockSpec((1, D), lambda i: (0, 0)),
            pl.BlockSpec(wqkv.shape, lambda i: (0, 0)),
            pl.BlockSpec(wo.shape, lambda i: (0, 0)),
        ],
        out_specs=pl.BlockSpec((_WIN, D), lambda i: (i, 0)),
        out_shape=jax.ShapeDtypeStruct((T, D), jnp.float32),
    )(x, ga.reshape(1, D), wqkv, wo)


def _heavy_body(x_ref, ga_ref, rq_ref, rk_ref, w_ref, wo_ref,
                eq_ref, oh_ref, ek_ref, vq_ref, vk_ref):
    x = x_ref[...]
    S, D = x.shape
    xn = _rms(x, ga_ref[...])
    sq = _fdot(xn, rq_ref[...])
    sk = _fdot(xn, rk_ref[...])
    _topk_into(sq, eq_ref, vq_ref, _K)
    _topk_into(sk, ek_ref, vk_ref, _K)
    Eq = eq_ref[...]
    Ek = ek_ref[...]
    xq = _xdot(Eq, xn)
    xk = _xdot(Ek, xn)
    qh = _bdot(xq, w_ref[:, :D])
    kh = _bdot(xk, w_ref[:, D:2 * D])
    vh = _bdot(xk, w_ref[:, 2 * D:]) * jax.nn.sigmoid(vk_ref[...])
    ah = _bdotT(qh, kh) / (D ** 0.5)
    iota_col = jax.lax.broadcasted_iota(jnp.int32, (S, 1), 0).astype(jnp.float32)
    iota_row = jax.lax.broadcasted_iota(jnp.int32, (1, S), 1).astype(jnp.float32)
    iq = _xdot(Eq, iota_col)
    ik = _xdotT(iota_row, Ek)
    ah = jax.nn.softmax(jnp.where(iq >= ik, ah, _NEG), axis=-1)
    oh_ref[...] = (_bdot(_bdot(ah, vh), wo_ref[...])
                   * jax.nn.sigmoid(vq_ref[...]))


def _heavy_attn(x, B, ga, rq, rk, w, wo):
    T, D = x.shape
    S = T // B
    return pl.pallas_call(
        _heavy_body,
        grid=(B,),
        in_specs=[
            pl.BlockSpec((S, D), lambda b: (b, 0)),
            pl.BlockSpec((1, D), lambda b: (0, 0)),
            pl.BlockSpec((D, 1), lambda b: (0, 0)),
            pl.BlockSpec((D, 1), lambda b: (0, 0)),
            pl.BlockSpec(w.shape, lambda b: (0, 0)),
            pl.BlockSpec(wo.shape, lambda b: (0, 0)),
        ],
        out_specs=[
            pl.BlockSpec((_K, S), lambda b: (b, 0)),
            pl.BlockSpec((_K, D), lambda b: (b, 0)),
        ],
        out_shape=[
            jax.ShapeDtypeStruct((B * _K, S), jnp.float32),
            jax.ShapeDtypeStruct((B * _K, D), jnp.float32),
        ],
        scratch_shapes=[
            pltpu.VMEM((_K, S), jnp.float32),
            pltpu.VMEM((_K, 1), jnp.float32),
            pltpu.VMEM((_K, 1), jnp.float32),
        ],
    )(x, ga.reshape(1, D), rq.reshape(D, 1), rk.reshape(D, 1), w, wo)


def _upd_route_body(xl_ref, eq_ref, oh_ref, gc_ref, rc_ref,
                    x1_ref, ec_ref, vc_ref, xc_ref):
    x1 = xl_ref[...] + _xdot0(eq_ref[...], oh_ref[...])
    x1_ref[...] = x1
    xn = _rms(x1, gc_ref[...])
    sc = _fdot(xn, rc_ref[...])
    _topk_into(sc, ec_ref, vc_ref, _K)
    xc_ref[...] = _xdot(ec_ref[...], xn)


def _upd_route(xl, Eq, oh, B, gc, rc):
    T, D = xl.shape
    S = T // B
    return pl.pallas_call(
        _upd_route_body,
        grid=(B,),
        in_specs=[
            pl.BlockSpec((S, D), lambda b: (b, 0)),
            pl.BlockSpec((_K, S), lambda b: (b, 0)),
            pl.BlockSpec((_K, D), lambda b: (b, 0)),
            pl.BlockSpec((1, D), lambda b: (0, 0)),
            pl.BlockSpec((D, 1), lambda b: (0, 0)),
        ],
        out_specs=[
            pl.BlockSpec((S, D), lambda b: (b, 0)),
            pl.BlockSpec((_K, S), lambda b: (b, 0)),
            pl.BlockSpec((_K, 1), lambda b: (b, 0)),
            pl.BlockSpec((_K, D), lambda b: (b, 0)),
        ],
        out_shape=[
            jax.ShapeDtypeStruct((T, D), jnp.float32),
            jax.ShapeDtypeStruct((B * _K, S), jnp.float32),
            jax.ShapeDtypeStruct((B * _K, 1), jnp.float32),
            jax.ShapeDtypeStruct((B * _K, D), jnp.float32),
        ],
    )(xl, Eq, oh, gc.reshape(1, D), rc.reshape(D, 1))


def _cross_body(xc_ref, kvc_ref, wq_ref, woc_ref, vc_ref, oc_ref):
    xc = xc_ref[...]
    D = xc.shape[1]
    qc = _bdot(xc, wq_ref[...])
    a = _bdotT(qc, kvc_ref[:, :D]) / (D ** 0.5)
    ac = jax.nn.softmax(a, axis=-1)
    u = _bdot(ac, kvc_ref[:, D:])
    oc_ref[...] = _bdot(u, woc_ref[...]) * jax.nn.sigmoid(vc_ref[...])


def _cross_attn(xc, kvc, B, wq, woc, vc):
    TK, D = xc.shape
    SE = kvc.shape[0] // B
    return pl.pallas_call(
        _cross_body,
        grid=(B,),
        in_specs=[
            pl.BlockSpec((_K, D), lambda b: (b, 0)),
            pl.BlockSpec((SE, 2 * D), lambda b: (b, 0)),
            pl.BlockSpec(wq.shape, lambda b: (0, 0)),
            pl.BlockSpec(woc.shape, lambda b: (0, 0)),
            pl.BlockSpec((_K, 1), lambda b: (b, 0)),
        ],
        out_specs=pl.BlockSpec((_K, D), lambda b: (b, 0)),
        out_shape=jax.ShapeDtypeStruct((B * _K, D), jnp.float32),
    )(xc, kvc, wq, woc, vc)


def _upd_ff_body(x1_ref, oc_ref, ec_ref, gf_ref, rf_ref,
                 x2_ref, ef_ref, vf_ref, xf_ref):
    x2 = x1_ref[...] + _xdot0(ec_ref[...], oc_ref[...])
    x2_ref[...] = x2
    xn = _rms(x2, gf_ref[...])
    sf = _fdot(xn, rf_ref[...])
    _topk_into(sf, ef_ref, vf_ref, _K)
    xf_ref[...] = _xdot(ef_ref[...], xn)


def _upd_ff(x1, oc, Ec, B, gf, rf):
    T, D = x1.shape
    S = T // B
    return pl.pallas_call(
        _upd_ff_body,
        grid=(B,),
        in_specs=[
            pl.BlockSpec((S, D), lambda b: (b, 0)),
            pl.BlockSpec((_K, D), lambda b: (b, 0)),
            pl.BlockSpec((_K, S), lambda b: (b, 0)),
            pl.BlockSpec((1, D), lambda b: (0, 0)),
            pl.BlockSpec((D, 1), lambda b: (0, 0)),
        ],
        out_specs=[
            pl.BlockSpec((S, D), lambda b: (b, 0)),
            pl.BlockSpec((_K, S), lambda b: (b, 0)),
            pl.BlockSpec((_K, 1), lambda b: (b, 0)),
            pl.BlockSpec((_K, D), lambda b: (b, 0)),
        ],
        out_shape=[
            jax.ShapeDtypeStruct((T, D), jnp.float32),
            jax.ShapeDtypeStruct((B * _K, S), jnp.float32),
            jax.ShapeDtypeStruct((B * _K, 1), jnp.float32),
            jax.ShapeDtypeStruct((B * _K, D), jnp.float32),
        ],
    )(x1, oc, Ec, gf.reshape(1, D), rf.reshape(D, 1))


def _hff_body(xf_ref, w1_ref, w2_ref, vf_ref, hf_ref):
    h = jax.nn.gelu(_bdot(xf_ref[...], w1_ref[...]))
    hf_ref[...] = _bdot(h, w2_ref[...]) * jax.nn.sigmoid(vf_ref[...])


def _heavy_ff(xf, vf, B, w1, w2):
    TK, D = xf.shape
    return pl.pallas_call(
        _hff_body,
        grid=(B,),
        in_specs=[
            pl.BlockSpec((_K, D), lambda b: (b, 0)),
            pl.BlockSpec(w1.shape, lambda b: (0, 0)),
            pl.BlockSpec(w2.shape, lambda b: (0, 0)),
            pl.BlockSpec((_K, 1), lambda b: (b, 0)),
        ],
        out_specs=pl.BlockSpec((_K, D), lambda b: (b, 0)),
        out_shape=jax.ShapeDtypeStruct((B * _K, D), jnp.float32),
    )(xf, w1, w2, vf)


def _final_body(x2_ref, gf_ref, w1_ref, w2_ref, ef_ref, hf_ref, o_ref):
    x2 = x2_ref[...]
    xn = _rms(x2, gf_ref[...])
    lf = _bdot(jax.nn.gelu(_bdot(xn, w1_ref[...])), w2_ref[...])
    o_ref[...] = x2 + lf + _xdot0(ef_ref[...], hf_ref[...])


def _final_ff(x2, Ef, hf, B, gf, w1, w2):
    T, D = x2.shape
    S = T // B
    nt = S // _TB
    return pl.pallas_call(
        _final_body,
        grid=(B, nt),
        in_specs=[
            pl.BlockSpec((_TB, D), lambda b, n: (b * nt + n, 0)),
            pl.BlockSpec((1, D), lambda b, n: (0, 0)),
            pl.BlockSpec(w1.shape, lambda b, n: (0, 0)),
            pl.BlockSpec(w2.shape, lambda b, n: (0, 0)),
            pl.BlockSpec((_K, _TB), lambda b, n: (b, n)),
            pl.BlockSpec((_K, D), lambda b, n: (b, 0)),
        ],
        out_specs=pl.BlockSpec((_TB, D), lambda b, n: (b * nt + n, 0)),
        out_shape=jax.ShapeDtypeStruct((T, D), jnp.float32),
    )(x2, gf.reshape(1, D), w1, w2, Ef, hf)


def kernel(input_ids, encoder_hidden_states, embed, route_q, route_kv,
           route_c, route_ff, Wqkv_l, Wo_l, Wqkv_h, Wo_h, Wq_c, Wkv_c, Wo_c,
           W1_lf, W2_lf, W1_hf, W2_hf, g_a, g_c, g_f):
    B, S = input_ids.shape
    V, D = embed.shape
    L = route_q.shape[0]
    SE = encoder_hidden_states.shape[1]
    bf = jnp.bfloat16
    ids = input_ids.reshape(-1).astype(jnp.int32)
    x = _embed_gather(ids, embed)
    enc_bf = encoder_hidden_states.reshape(B * SE, D).astype(bf)
    for l in range(L):
        kvc = _kvc_proj(enc_bf, Wkv_c[l].astype(bf))
        xl = _light_attn(x, g_a[l], Wqkv_l[l].astype(bf), Wo_l[l].astype(bf))
        Eq, oh = _heavy_attn(x, B, g_a[l], route_q[l], route_kv[l],
                             Wqkv_h[l].astype(bf), Wo_h[l].astype(bf))
        x1, Ec, vc, xc = _upd_route(xl, Eq, oh, B, g_c[l], route_c[l])
        oc = _cross_attn(xc, kvc, B, Wq_c[l].astype(bf), Wo_c[l].astype(bf),
                         vc)
        x2, Ef, vf, xf = _upd_ff(x1, oc, Ec, B, g_f[l], route_ff[l])
        hf = _heavy_ff(xf, vf, B, W1_hf[l].astype(bf), W2_hf[l].astype(bf))
        x = _final_ff(x2, Ef, hf, B, g_f[l], W1_lf[l].astype(bf),
                      W2_lf[l].astype(bf))
    return x.reshape(B, S, D)


# speedup vs baseline: 1.2817x; 1.2817x over previous
import functools

import jax
import jax.numpy as jnp
from jax.experimental import pallas as pl
from jax.experimental.pallas import tpu as pltpu

_K = 32
_WIN = 128
_TB = 256
_EB = 512
_NEG = -1e9
_R = 8

_HI = jax.lax.Precision.HIGHEST


def _bdot(a, b):
    return jax.lax.dot_general(
        a.astype(jnp.bfloat16), b.astype(jnp.bfloat16),
        (((1,), (0,)), ((), ())), preferred_element_type=jnp.float32)


def _bdotT(a, b):
    return jax.lax.dot_general(
        a.astype(jnp.bfloat16), b.astype(jnp.bfloat16),
        (((1,), (1,)), ((), ())), preferred_element_type=jnp.float32)


def _fdot(a, b):
    return jax.lax.dot_general(a, b, (((1,), (0,)), ((), ())),
                               preferred_element_type=jnp.float32)


def _xdot(a, b):
    return jax.lax.dot_general(a, b, (((1,), (0,)), ((), ())),
                               preferred_element_type=jnp.float32,
                               precision=_HI)


def _xdotT(a, b):
    return jax.lax.dot_general(a, b, (((1,), (1,)), ((), ())),
                               preferred_element_type=jnp.float32,
                               precision=_HI)


def _xdot0(a, b):
    return jax.lax.dot_general(a, b, (((0,), (0,)), ((), ())),
                               preferred_element_type=jnp.float32,
                               precision=_HI)


def _rms(x, g):
    return x * g / jnp.sqrt(jnp.mean(x * x, axis=-1, keepdims=True) + 1e-6)


def _topk_into(s_col, e_ref, v_ref, k):
    S = s_col.shape[0]
    iota_row = jax.lax.broadcasted_iota(jnp.int32, (1, S), 1).astype(jnp.float32)

    def body(j, s):
        m = jnp.max(s)
        idx = jnp.min(jnp.where(s == m, iota_row, jnp.float32(S)))
        e_ref[pl.ds(j, 1), :] = (iota_row == idx).astype(jnp.float32)
        v_ref[pl.ds(j, 1), :] = jnp.reshape(m, (1, 1))
        return jnp.where(iota_row == idx, -jnp.inf, s)

    jax.lax.fori_loop(0, k, body, jnp.reshape(s_col, (1, S)))


def _embed_body(ids_ref, *refs):
    out_ref = refs[-1]
    for j in range(_R):
        out_ref[j, :] = refs[j][0, 0, :]


def _embed_gather(ids_flat, embed):
    T = ids_flat.shape[0]
    V, D = embed.shape
    embed3 = embed.reshape(V, 1, D)

    def imap(j, i, ids):
        return (ids[i * _R + j], 0, 0)

    return pl.pallas_call(
        _embed_body,
        grid_spec=pltpu.PrefetchScalarGridSpec(
            num_scalar_prefetch=1,
            grid=(T // _R,),
            in_specs=[pl.BlockSpec((1, 1, D), functools.partial(imap, j))
                      for j in range(_R)],
            out_specs=pl.BlockSpec((_R, D), lambda i, ids: (i, 0)),
        ),
        out_shape=jax.ShapeDtypeStruct((T, D), jnp.float32),
    )(ids_flat, *([embed3] * _R))


def _kvc_body(enc_ref, wkv_ref, o_ref):
    o_ref[...] = _bdot(enc_ref[...], wkv_ref[...]).astype(jnp.bfloat16)


def _kvc_proj(enc_bf, wkv_bf):
    T, D = enc_bf.shape
    D2 = wkv_bf.shape[1]
    eb = min(_EB, T)
    return pl.pallas_call(
        _kvc_body,
        grid=(T // eb,),
        in_specs=[
            pl.BlockSpec((eb, D), lambda i: (i, 0)),
            pl.BlockSpec(wkv_bf.shape, lambda i: (0, 0)),
        ],
        out_specs=pl.BlockSpec((eb, D2), lambda i: (i, 0)),
        out_shape=jax.ShapeDtypeStruct((T, D2), jnp.bfloat16),
    )(enc_bf, wkv_bf)


def _light_body(x_ref, ga_ref, wqkv_ref, wo_ref, o_ref):
    x = x_ref[...]
    dl = wo_ref.shape[0]
    xn = _rms(x, ga_ref[...])
    qkv = _bdot(xn, wqkv_ref[...])
    q = qkv[:, :dl]
    k = qkv[:, dl:2 * dl]
    v = qkv[:, 2 * dl:]
    a = _bdotT(q, k) / (dl ** 0.5)
    W = x.shape[0]
    r = jax.lax.broadcasted_iota(jnp.int32, (W, W), 0)
    c = jax.lax.broadcasted_iota(jnp.int32, (W, W), 1)
    a = jax.nn.softmax(jnp.where(r >= c, a, _NEG), axis=-1)
    o_ref[...] = x + _bdot(_bdot(a, v), wo_ref[...])


def _light_attn(x, ga, wqkv, wo):
    T, D = x.shape
    return pl.pallas_call(
        _light_body,
        grid=(T // _WIN,),
        in_specs=[
            pl.BlockSpec((_WIN, D), lambda i: (i, 0)),
            pl.BlockSpec((1, D), lambda i: (0, 0)),
            pl.BlockSpec(wqkv.shape, lambda i: (0, 0)),
            pl.BlockSpec(wo.shape, lambda i: (0, 0)),
        ],
        out_specs=pl.BlockSpec((_WIN, D), lambda i: (i, 0)),
        out_shape=jax.ShapeDtypeStruct((T, D), jnp.float32),
    )(x, ga.reshape(1, D), wqkv, wo)


def _heavy_body(x_ref, ga_ref, rq_ref, rk_ref, w_ref, wo_ref,
                eq_ref, oh_ref, ek_ref, vq_ref, vk_ref):
    x = x_ref[...]
    S, D = x.shape
    xn = _rms(x, ga_ref[...])
    sq = _fdot(xn, rq_ref[...])
    sk = _fdot(xn, rk_ref[...])
    _topk_into(sq, eq_ref, vq_ref, _K)
    _topk_into(sk, ek_ref, vk_ref, _K)
    Eq = eq_ref[...]
    Ek = ek_ref[...]
    xq = _xdot(Eq, xn)
    xk = _xdot(Ek, xn)
    qh = _bdot(xq, w_ref[:, :D])
    kh = _bdot(xk, w_ref[:, D:2 * D])
    vh = _bdot(xk, w_ref[:, 2 * D:]) * jax.nn.sigmoid(vk_ref[...])
    ah = _bdotT(qh, kh) / (D ** 0.5)
    iota_col = jax.lax.broadcasted_iota(jnp.int32, (S, 1), 0).astype(jnp.float32)
    iota_row = jax.lax.broadcasted_iota(jnp.int32, (1, S), 1).astype(jnp.float32)
    iq = _xdot(Eq, iota_col)
    ik = _xdotT(iota_row, Ek)
    ah = jax.nn.softmax(jnp.where(iq >= ik, ah, _NEG), axis=-1)
    oh_ref[...] = (_bdot(_bdot(ah, vh), wo_ref[...])
                   * jax.nn.sigmoid(vq_ref[...]))


def _heavy_attn(x, B, ga, rq, rk, w, wo):
    T, D = x.shape
    S = T // B
    return pl.pallas_call(
        _heavy_body,
        grid=(B,),
        in_specs=[
            pl.BlockSpec((S, D), lambda b: (b, 0)),
            pl.BlockSpec((1, D), lambda b: (0, 0)),
            pl.BlockSpec((D, 1), lambda b: (0, 0)),
            pl.BlockSpec((D, 1), lambda b: (0, 0)),
            pl.BlockSpec(w.shape, lambda b: (0, 0)),
            pl.BlockSpec(wo.shape, lambda b: (0, 0)),
        ],
        out_specs=[
            pl.BlockSpec((_K, S), lambda b: (b, 0)),
            pl.BlockSpec((_K, D), lambda b: (b, 0)),
        ],
        out_shape=[
            jax.ShapeDtypeStruct((B * _K, S), jnp.float32),
            jax.ShapeDtypeStruct((B * _K, D), jnp.float32),
        ],
        scratch_shapes=[
            pltpu.VMEM((_K, S), jnp.float32),
            pltpu.VMEM((_K, 1), jnp.float32),
            pltpu.VMEM((_K, 1), jnp.float32),
        ],
    )(x, ga.reshape(1, D), rq.reshape(D, 1), rk.reshape(D, 1), w, wo)


def _upd_route_body(xl_ref, eq_ref, oh_ref, gc_ref, rc_ref,
                    x1_ref, ec_ref, vc_ref, xc_ref):
    x1 = xl_ref[...] + _xdot0(eq_ref[...], oh_ref[...])
    x1_ref[...] = x1
    xn = _rms(x1, gc_ref[...])
    sc = _fdot(xn, rc_ref[...])
    _topk_into(sc, ec_ref, vc_ref, _K)
    xc_ref[...] = _xdot(ec_ref[...], xn)


def _upd_route(xl, Eq, oh, B, gc, rc):
    T, D = xl.shape
    S = T // B
    return pl.pallas_call(
        _upd_route_body,
        grid=(B,),
        in_specs=[
            pl.BlockSpec((S, D), lambda b: (b, 0)),
            pl.BlockSpec((_K, S), lambda b: (b, 0)),
            pl.BlockSpec((_K, D), lambda b: (b, 0)),
            pl.BlockSpec((1, D), lambda b: (0, 0)),
            pl.BlockSpec((D, 1), lambda b: (0, 0)),
        ],
        out_specs=[
            pl.BlockSpec((S, D), lambda b: (b, 0)),
            pl.BlockSpec((_K, S), lambda b: (b, 0)),
            pl.BlockSpec((_K, 1), lambda b: (b, 0)),
            pl.BlockSpec((_K, D), lambda b: (b, 0)),
        ],
        out_shape=[
            jax.ShapeDtypeStruct((T, D), jnp.float32),
            jax.ShapeDtypeStruct((B * _K, S), jnp.float32),
            jax.ShapeDtypeStruct((B * _K, 1), jnp.float32),
            jax.ShapeDtypeStruct((B * _K, D), jnp.float32),
        ],
    )(xl, Eq, oh, gc.reshape(1, D), rc.reshape(D, 1))


def _cross_body(xc_ref, kvc_ref, wq_ref, woc_ref, vc_ref, oc_ref):
    xc = xc_ref[...]
    D = xc.shape[1]
    qc = _bdot(xc, wq_ref[...])
    a = _bdotT(qc, kvc_ref[:, :D]) / (D ** 0.5)
    ac = jax.nn.softmax(a, axis=-1)
    u = _bdot(ac, kvc_ref[:, D:])
    oc_ref[...] = _bdot(u, woc_ref[...]) * jax.nn.sigmoid(vc_ref[...])


def _cross_attn(xc, kvc, B, wq, woc, vc):
    TK, D = xc.shape
    SE = kvc.shape[0] // B
    return pl.pallas_call(
        _cross_body,
        grid=(B,),
        in_specs=[
            pl.BlockSpec((_K, D), lambda b: (b, 0)),
            pl.BlockSpec((SE, 2 * D), lambda b: (b, 0)),
            pl.BlockSpec(wq.shape, lambda b: (0, 0)),
            pl.BlockSpec(woc.shape, lambda b: (0, 0)),
            pl.BlockSpec((_K, 1), lambda b: (b, 0)),
        ],
        out_specs=pl.BlockSpec((_K, D), lambda b: (b, 0)),
        out_shape=jax.ShapeDtypeStruct((B * _K, D), jnp.float32),
    )(xc, kvc, wq, woc, vc)


def _upd_ff_body(x1_ref, oc_ref, ec_ref, gf_ref, rf_ref,
                 x2_ref, ef_ref, vf_ref, xf_ref):
    x2 = x1_ref[...] + _xdot0(ec_ref[...], oc_ref[...])
    x2_ref[...] = x2
    xn = _rms(x2, gf_ref[...])
    sf = _fdot(xn, rf_ref[...])
    _topk_into(sf, ef_ref, vf_ref, _K)
    xf_ref[...] = _xdot(ef_ref[...], xn)


def _upd_ff(x1, oc, Ec, B, gf, rf):
    T, D = x1.shape
    S = T // B
    return pl.pallas_call(
        _upd_ff_body,
        grid=(B,),
        in_specs=[
            pl.BlockSpec((S, D), lambda b: (b, 0)),
            pl.BlockSpec((_K, D), lambda b: (b, 0)),
            pl.BlockSpec((_K, S), lambda b: (b, 0)),
            pl.BlockSpec((1, D), lambda b: (0, 0)),
            pl.BlockSpec((D, 1), lambda b: (0, 0)),
        ],
        out_specs=[
            pl.BlockSpec((S, D), lambda b: (b, 0)),
            pl.BlockSpec((_K, S), lambda b: (b, 0)),
            pl.BlockSpec((_K, 1), lambda b: (b, 0)),
            pl.BlockSpec((_K, D), lambda b: (b, 0)),
        ],
        out_shape=[
            jax.ShapeDtypeStruct((T, D), jnp.float32),
            jax.ShapeDtypeStruct((B * _K, S), jnp.float32),
            jax.ShapeDtypeStruct((B * _K, 1), jnp.float32),
            jax.ShapeDtypeStruct((B * _K, D), jnp.float32),
        ],
    )(x1, oc, Ec, gf.reshape(1, D), rf.reshape(D, 1))


def _hff_body(xf_ref, w1_ref, w2_ref, vf_ref, hf_ref):
    h = jax.nn.gelu(_bdot(xf_ref[...], w1_ref[...]))
    hf_ref[...] = _bdot(h, w2_ref[...]) * jax.nn.sigmoid(vf_ref[...])


def _heavy_ff(xf, vf, B, w1, w2):
    TK, D = xf.shape
    return pl.pallas_call(
        _hff_body,
        grid=(B,),
        in_specs=[
            pl.BlockSpec((_K, D), lambda b: (b, 0)),
            pl.BlockSpec(w1.shape, lambda b: (0, 0)),
            pl.BlockSpec(w2.shape, lambda b: (0, 0)),
            pl.BlockSpec((_K, 1), lambda b: (b, 0)),
        ],
        out_specs=pl.BlockSpec((_K, D), lambda b: (b, 0)),
        out_shape=jax.ShapeDtypeStruct((B * _K, D), jnp.float32),
    )(xf, w1, w2, vf)


def _final_body(x2_ref, gf_ref, w1_ref, w2_ref, ef_ref, hf_ref, o_ref):
    x2 = x2_ref[...]
    xn = _rms(x2, gf_ref[...])
    lf = _bdot(jax.nn.gelu(_bdot(xn, w1_ref[...])), w2_ref[...])
    o_ref[...] = x2 + lf + _xdot0(ef_ref[...], hf_ref[...])


def _final_ff(x2, Ef, hf, B, gf, w1, w2):
    T, D = x2.shape
    S = T // B
    nt = S // _TB
    return pl.pallas_call(
        _final_body,
        grid=(B, nt),
        in_specs=[
            pl.BlockSpec((_TB, D), lambda b, n: (b * nt + n, 0)),
            pl.BlockSpec((1, D), lambda b, n: (0, 0)),
            pl.BlockSpec(w1.shape, lambda b, n: (0, 0)),
            pl.BlockSpec(w2.shape, lambda b, n: (0, 0)),
            pl.BlockSpec((_K, _TB), lambda b, n: (b, n)),
            pl.BlockSpec((_K, D), lambda b, n: (b, 0)),
        ],
        out_specs=pl.BlockSpec((_TB, D), lambda b, n: (b * nt + n, 0)),
        out_shape=jax.ShapeDtypeStruct((T, D), jnp.float32),
    )(x2, gf.reshape(1, D), w1, w2, Ef, hf)


def kernel(input_ids, encoder_hidden_states, embed, route_q, route_kv,
           route_c, route_ff, Wqkv_l, Wo_l, Wqkv_h, Wo_h, Wq_c, Wkv_c, Wo_c,
           W1_lf, W2_lf, W1_hf, W2_hf, g_a, g_c, g_f):
    B, S = input_ids.shape
    V, D = embed.shape
    L = route_q.shape[0]
    SE = encoder_hidden_states.shape[1]
    bf = jnp.bfloat16
    ids = input_ids.reshape(-1).astype(jnp.int32)
    x = _embed_gather(ids, embed)
    enc_bf = encoder_hidden_states.reshape(B * SE, D).astype(bf)
    for l in range(L):
        kvc = _kvc_proj(enc_bf, Wkv_c[l].astype(bf))
        xl = _light_attn(x, g_a[l], Wqkv_l[l].astype(bf), Wo_l[l].astype(bf))
        Eq, oh = _heavy_attn(x, B, g_a[l], route_q[l], route_kv[l],
                             Wqkv_h[l].astype(bf), Wo_h[l].astype(bf))
        x1, Ec, vc, xc = _upd_route(xl, Eq, oh, B, g_c[l], route_c[l])
        oc = _cross_attn(xc, kvc, B, Wq_c[l].astype(bf), Wo_c[l].astype(bf),
                         vc)
        x2, Ef, vf, xf = _upd_ff(x1, oc, Ec, B, g_f[l], route_ff[l])
        hf = _heavy_ff(xf, vf, B, W1_hf[l].astype(bf), W2_hf[l].astype(bf))
        x = _final_ff(x2, Ef, hf, B, g_f[l], W1_lf[l].astype(bf),
                      W2_lf[l].astype(bf))
    return x.reshape(B, S, D)


# speedup vs baseline: 1.5332x; 1.1962x over previous
import functools

import jax
import jax.numpy as jnp
from jax.experimental import pallas as pl
from jax.experimental.pallas import tpu as pltpu

_K = 32
_WIN = 128
_TB = 256
_EB = 512
_NEG = -1e9
_R = 16

_HI = jax.lax.Precision.HIGHEST


def _bdot(a, b):
    return jax.lax.dot_general(
        a.astype(jnp.bfloat16), b.astype(jnp.bfloat16),
        (((1,), (0,)), ((), ())), preferred_element_type=jnp.float32)


def _bdotT(a, b):
    return jax.lax.dot_general(
        a.astype(jnp.bfloat16), b.astype(jnp.bfloat16),
        (((1,), (1,)), ((), ())), preferred_element_type=jnp.float32)


def _fdot(a, b):
    return jax.lax.dot_general(a, b, (((1,), (0,)), ((), ())),
                               preferred_element_type=jnp.float32)


def _split3(y):
    h = y.astype(jnp.bfloat16)
    m = (y - h.astype(jnp.float32)).astype(jnp.bfloat16)
    lo = (y - h.astype(jnp.float32) - m.astype(jnp.float32)).astype(jnp.bfloat16)
    return h, m, lo


def _e3(a, b, dims):
    ab = a.astype(jnp.bfloat16)
    h, m, lo = _split3(b)
    dn = (dims, ((), ()))
    f = functools.partial(jax.lax.dot_general, preferred_element_type=jnp.float32)
    return (f(ab, h, dn) + f(ab, m, dn)) + f(ab, lo, dn)


def _xdot(a, b):
    return _e3(a, b, ((1,), (0,)))


def _xdotT(a, b):
    bb = b.astype(jnp.bfloat16)
    h, m, lo = _split3(a)
    dn = (((1,), (1,)), ((), ()))
    f = functools.partial(jax.lax.dot_general, preferred_element_type=jnp.float32)
    return (f(h, bb, dn) + f(m, bb, dn)) + f(lo, bb, dn)


def _xdot0(a, b):
    return _e3(a, b, ((0,), (0,)))


def _rms(x, g):
    return x * g / jnp.sqrt(jnp.mean(x * x, axis=-1, keepdims=True) + 1e-6)


def _topk_into(s_col, e_ref, v_ref, k):
    S = s_col.shape[0]
    iota_row = jax.lax.broadcasted_iota(jnp.int32, (1, S), 1).astype(jnp.float32)

    def body(j, s):
        m = jnp.max(s)
        idx = jnp.min(jnp.where(s == m, iota_row, jnp.float32(S)))
        e_ref[pl.ds(j, 1), :] = (iota_row == idx).astype(jnp.float32)
        v_ref[pl.ds(j, 1), :] = jnp.reshape(m, (1, 1))
        return jnp.where(iota_row == idx, -jnp.inf, s)

    jax.lax.fori_loop(0, k, body, jnp.reshape(s_col, (1, S)))


def _embed_body(ids_ref, *refs):
    out_ref = refs[-1]
    for j in range(_R):
        out_ref[j, :] = refs[j][0, 0, :]


def _embed_gather(ids_flat, embed):
    T = ids_flat.shape[0]
    V, D = embed.shape
    embed3 = embed.reshape(V, 1, D)

    def imap(j, i, ids):
        return (ids[i * _R + j], 0, 0)

    return pl.pallas_call(
        _embed_body,
        grid_spec=pltpu.PrefetchScalarGridSpec(
            num_scalar_prefetch=1,
            grid=(T // _R,),
            in_specs=[pl.BlockSpec((1, 1, D), functools.partial(imap, j))
                      for j in range(_R)],
            out_specs=pl.BlockSpec((_R, D), lambda i, ids: (i, 0)),
        ),
        out_shape=jax.ShapeDtypeStruct((T, D), jnp.float32),
    )(ids_flat, *([embed3] * _R))


def _kvc_body(enc_ref, wkv_ref, o_ref):
    o_ref[...] = _bdot(enc_ref[...], wkv_ref[...]).astype(jnp.bfloat16)


def _kvc_proj(enc_bf, wkv_bf):
    T, D = enc_bf.shape
    D2 = wkv_bf.shape[1]
    eb = min(_EB, T)
    return pl.pallas_call(
        _kvc_body,
        grid=(T // eb,),
        in_specs=[
            pl.BlockSpec((eb, D), lambda i: (i, 0)),
            pl.BlockSpec(wkv_bf.shape, lambda i: (0, 0)),
        ],
        out_specs=pl.BlockSpec((eb, D2), lambda i: (i, 0)),
        out_shape=jax.ShapeDtypeStruct((T, D2), jnp.bfloat16),
    )(enc_bf, wkv_bf)


def _light_body(x_ref, ga_ref, wqkv_ref, wo_ref, o_ref):
    x = x_ref[...]
    dl = wo_ref.shape[0]
    xn = _rms(x, ga_ref[...])
    qkv = _bdot(xn, wqkv_ref[...])
    q = qkv[:, :dl]
    k = qkv[:, dl:2 * dl]
    v = qkv[:, 2 * dl:]
    a = _bdotT(q, k) / (dl ** 0.5)
    W = x.shape[0]
    r = jax.lax.broadcasted_iota(jnp.int32, (W, W), 0)
    c = jax.lax.broadcasted_iota(jnp.int32, (W, W), 1)
    a = jax.nn.softmax(jnp.where(r >= c, a, _NEG), axis=-1)
    o_ref[...] = x + _bdot(_bdot(a, v), wo_ref[...])


def _light_attn(x, ga, wqkv, wo):
    T, D = x.shape
    return pl.pallas_call(
        _light_body,
        grid=(T // _WIN,),
        in_specs=[
            pl.BlockSpec((_WIN, D), lambda i: (i, 0)),
            pl.BlockSpec((1, D), lambda i: (0, 0)),
            pl.BlockSpec(wqkv.shape, lambda i: (0, 0)),
            pl.BlockSpec(wo.shape, lambda i: (0, 0)),
        ],
        out_specs=pl.BlockSpec((_WIN, D), lambda i: (i, 0)),
        out_shape=jax.ShapeDtypeStruct((T, D), jnp.float32),
    )(x, ga.reshape(1, D), wqkv, wo)


def _heavy_body(x_ref, ga_ref, rq_ref, rk_ref, w_ref, wo_ref,
                eq_ref, oh_ref, ek_ref, vq_ref, vk_ref):
    x = x_ref[...]
    S, D = x.shape
    xn = _rms(x, ga_ref[...])
    sq = _fdot(xn, rq_ref[...])
    sk = _fdot(xn, rk_ref[...])
    _topk_into(sq, eq_ref, vq_ref, _K)
    _topk_into(sk, ek_ref, vk_ref, _K)
    Eq = eq_ref[...]
    Ek = ek_ref[...]
    xq = _xdot(Eq, xn)
    xk = _xdot(Ek, xn)
    qh = _bdot(xq, w_ref[:, :D])
    kh = _bdot(xk, w_ref[:, D:2 * D])
    vh = _bdot(xk, w_ref[:, 2 * D:]) * jax.nn.sigmoid(vk_ref[...])
    ah = _bdotT(qh, kh) / (D ** 0.5)
    iota_col = jax.lax.broadcasted_iota(jnp.int32, (S, 1), 0).astype(jnp.float32)
    iota_row = jax.lax.broadcasted_iota(jnp.int32, (1, S), 1).astype(jnp.float32)
    iq = _xdot(Eq, iota_col)
    ik = _xdotT(iota_row, Ek)
    ah = jax.nn.softmax(jnp.where(iq >= ik, ah, _NEG), axis=-1)
    oh_ref[...] = (_bdot(_bdot(ah, vh), wo_ref[...])
                   * jax.nn.sigmoid(vq_ref[...]))


def _heavy_attn(x, B, ga, rq, rk, w, wo):
    T, D = x.shape
    S = T // B
    return pl.pallas_call(
        _heavy_body,
        grid=(B,),
        in_specs=[
            pl.BlockSpec((S, D), lambda b: (b, 0)),
            pl.BlockSpec((1, D), lambda b: (0, 0)),
            pl.BlockSpec((D, 1), lambda b: (0, 0)),
            pl.BlockSpec((D, 1), lambda b: (0, 0)),
            pl.BlockSpec(w.shape, lambda b: (0, 0)),
            pl.BlockSpec(wo.shape, lambda b: (0, 0)),
        ],
        out_specs=[
            pl.BlockSpec((_K, S), lambda b: (b, 0)),
            pl.BlockSpec((_K, D), lambda b: (b, 0)),
        ],
        out_shape=[
            jax.ShapeDtypeStruct((B * _K, S), jnp.float32),
            jax.ShapeDtypeStruct((B * _K, D), jnp.float32),
        ],
        scratch_shapes=[
            pltpu.VMEM((_K, S), jnp.float32),
            pltpu.VMEM((_K, 1), jnp.float32),
            pltpu.VMEM((_K, 1), jnp.float32),
        ],
    )(x, ga.reshape(1, D), rq.reshape(D, 1), rk.reshape(D, 1), w, wo)


def _upd_route_body(xl_ref, eq_ref, oh_ref, gc_ref, rc_ref,
                    x1_ref, ec_ref, vc_ref, xc_ref):
    x1 = xl_ref[...] + _xdot0(eq_ref[...], oh_ref[...])
    x1_ref[...] = x1
    xn = _rms(x1, gc_ref[...])
    sc = _fdot(xn, rc_ref[...])
    _topk_into(sc, ec_ref, vc_ref, _K)
    xc_ref[...] = _xdot(ec_ref[...], xn)


def _upd_route(xl, Eq, oh, B, gc, rc):
    T, D = xl.shape
    S = T // B
    return pl.pallas_call(
        _upd_route_body,
        grid=(B,),
        in_specs=[
            pl.BlockSpec((S, D), lambda b: (b, 0)),
            pl.BlockSpec((_K, S), lambda b: (b, 0)),
            pl.BlockSpec((_K, D), lambda b: (b, 0)),
            pl.BlockSpec((1, D), lambda b: (0, 0)),
            pl.BlockSpec((D, 1), lambda b: (0, 0)),
        ],
        out_specs=[
            pl.BlockSpec((S, D), lambda b: (b, 0)),
            pl.BlockSpec((_K, S), lambda b: (b, 0)),
            pl.BlockSpec((_K, 1), lambda b: (b, 0)),
            pl.BlockSpec((_K, D), lambda b: (b, 0)),
        ],
        out_shape=[
            jax.ShapeDtypeStruct((T, D), jnp.float32),
            jax.ShapeDtypeStruct((B * _K, S), jnp.float32),
            jax.ShapeDtypeStruct((B * _K, 1), jnp.float32),
            jax.ShapeDtypeStruct((B * _K, D), jnp.float32),
        ],
    )(xl, Eq, oh, gc.reshape(1, D), rc.reshape(D, 1))


def _cross_body(xc_ref, kvc_ref, wq_ref, woc_ref, vc_ref, oc_ref):
    xc = xc_ref[...]
    D = xc.shape[1]
    qc = _bdot(xc, wq_ref[...])
    a = _bdotT(qc, kvc_ref[:, :D]) / (D ** 0.5)
    ac = jax.nn.softmax(a, axis=-1)
    u = _bdot(ac, kvc_ref[:, D:])
    oc_ref[...] = _bdot(u, woc_ref[...]) * jax.nn.sigmoid(vc_ref[...])


def _cross_attn(xc, kvc, B, wq, woc, vc):
    TK, D = xc.shape
    SE = kvc.shape[0] // B
    return pl.pallas_call(
        _cross_body,
        grid=(B,),
        in_specs=[
            pl.BlockSpec((_K, D), lambda b: (b, 0)),
            pl.BlockSpec((SE, 2 * D), lambda b: (b, 0)),
            pl.BlockSpec(wq.shape, lambda b: (0, 0)),
            pl.BlockSpec(woc.shape, lambda b: (0, 0)),
            pl.BlockSpec((_K, 1), lambda b: (b, 0)),
        ],
        out_specs=pl.BlockSpec((_K, D), lambda b: (b, 0)),
        out_shape=jax.ShapeDtypeStruct((B * _K, D), jnp.float32),
    )(xc, kvc, wq, woc, vc)


def _upd_ff_body(x1_ref, oc_ref, ec_ref, gf_ref, rf_ref,
                 x2_ref, ef_ref, vf_ref, xf_ref):
    x2 = x1_ref[...] + _xdot0(ec_ref[...], oc_ref[...])
    x2_ref[...] = x2
    xn = _rms(x2, gf_ref[...])
    sf = _fdot(xn, rf_ref[...])
    _topk_into(sf, ef_ref, vf_ref, _K)
    xf_ref[...] = _xdot(ef_ref[...], xn)


def _upd_ff(x1, oc, Ec, B, gf, rf):
    T, D = x1.shape
    S = T // B
    return pl.pallas_call(
        _upd_ff_body,
        grid=(B,),
        in_specs=[
            pl.BlockSpec((S, D), lambda b: (b, 0)),
            pl.BlockSpec((_K, D), lambda b: (b, 0)),
            pl.BlockSpec((_K, S), lambda b: (b, 0)),
            pl.BlockSpec((1, D), lambda b: (0, 0)),
            pl.BlockSpec((D, 1), lambda b: (0, 0)),
        ],
        out_specs=[
            pl.BlockSpec((S, D), lambda b: (b, 0)),
            pl.BlockSpec((_K, S), lambda b: (b, 0)),
            pl.BlockSpec((_K, 1), lambda b: (b, 0)),
            pl.BlockSpec((_K, D), lambda b: (b, 0)),
        ],
        out_shape=[
            jax.ShapeDtypeStruct((T, D), jnp.float32),
            jax.ShapeDtypeStruct((B * _K, S), jnp.float32),
            jax.ShapeDtypeStruct((B * _K, 1), jnp.float32),
            jax.ShapeDtypeStruct((B * _K, D), jnp.float32),
        ],
    )(x1, oc, Ec, gf.reshape(1, D), rf.reshape(D, 1))


def _hff_body(xf_ref, w1_ref, w2_ref, vf_ref, hf_ref):
    h = jax.nn.gelu(_bdot(xf_ref[...], w1_ref[...]))
    hf_ref[...] = _bdot(h, w2_ref[...]) * jax.nn.sigmoid(vf_ref[...])


def _heavy_ff(xf, vf, B, w1, w2):
    TK, D = xf.shape
    return pl.pallas_call(
        _hff_body,
        grid=(B,),
        in_specs=[
            pl.BlockSpec((_K, D), lambda b: (b, 0)),
            pl.BlockSpec(w1.shape, lambda b: (0, 0)),
            pl.BlockSpec(w2.shape, lambda b: (0, 0)),
            pl.BlockSpec((_K, 1), lambda b: (b, 0)),
        ],
        out_specs=pl.BlockSpec((_K, D), lambda b: (b, 0)),
        out_shape=jax.ShapeDtypeStruct((B * _K, D), jnp.float32),
    )(xf, w1, w2, vf)


def _final_body(x2_ref, gf_ref, w1_ref, w2_ref, ef_ref, hf_ref, o_ref):
    x2 = x2_ref[...]
    xn = _rms(x2, gf_ref[...])
    lf = _bdot(jax.nn.gelu(_bdot(xn, w1_ref[...])), w2_ref[...])
    o_ref[...] = x2 + lf + _xdot0(ef_ref[...], hf_ref[...])


def _final_ff(x2, Ef, hf, B, gf, w1, w2):
    T, D = x2.shape
    S = T // B
    nt = S // _TB
    return pl.pallas_call(
        _final_body,
        grid=(B, nt),
        in_specs=[
            pl.BlockSpec((_TB, D), lambda b, n: (b * nt + n, 0)),
            pl.BlockSpec((1, D), lambda b, n: (0, 0)),
            pl.BlockSpec(w1.shape, lambda b, n: (0, 0)),
            pl.BlockSpec(w2.shape, lambda b, n: (0, 0)),
            pl.BlockSpec((_K, _TB), lambda b, n: (b, n)),
            pl.BlockSpec((_K, D), lambda b, n: (b, 0)),
        ],
        out_specs=pl.BlockSpec((_TB, D), lambda b, n: (b * nt + n, 0)),
        out_shape=jax.ShapeDtypeStruct((T, D), jnp.float32),
    )(x2, gf.reshape(1, D), w1, w2, Ef, hf)


def kernel(input_ids, encoder_hidden_states, embed, route_q, route_kv,
           route_c, route_ff, Wqkv_l, Wo_l, Wqkv_h, Wo_h, Wq_c, Wkv_c, Wo_c,
           W1_lf, W2_lf, W1_hf, W2_hf, g_a, g_c, g_f):
    B, S = input_ids.shape
    V, D = embed.shape
    L = route_q.shape[0]
    SE = encoder_hidden_states.shape[1]
    bf = jnp.bfloat16
    ids = input_ids.reshape(-1).astype(jnp.int32)
    x = _embed_gather(ids, embed)
    enc_bf = encoder_hidden_states.reshape(B * SE, D).astype(bf)
    for l in range(L):
        kvc = _kvc_proj(enc_bf, Wkv_c[l].astype(bf))
        xl = _light_attn(x, g_a[l], Wqkv_l[l].astype(bf), Wo_l[l].astype(bf))
        Eq, oh = _heavy_attn(x, B, g_a[l], route_q[l], route_kv[l],
                             Wqkv_h[l].astype(bf), Wo_h[l].astype(bf))
        x1, Ec, vc, xc = _upd_route(xl, Eq, oh, B, g_c[l], route_c[l])
        oc = _cross_attn(xc, kvc, B, Wq_c[l].astype(bf), Wo_c[l].astype(bf),
                         vc)
        x2, Ef, vf, xf = _upd_ff(x1, oc, Ec, B, g_f[l], route_ff[l])
        hf = _heavy_ff(xf, vf, B, W1_hf[l].astype(bf), W2_hf[l].astype(bf))
        x = _final_ff(x2, Ef, hf, B, g_f[l], W1_lf[l].astype(bf),
                      W2_lf[l].astype(bf))
    return x.reshape(B, S, D)
